# Initial kernel scaffold; baseline (speedup 1.0000x reference)
import jax
import jax.numpy as jnp
from jax.experimental import pallas as pl


def kernel(feat_in, tar_candidate, W1p, b1p, gp, bep, W2p, b2p, W1m, b1m, gm, bem, W2m, b2m):
    raise NotImplementedError("write your pallas kernel here")


# speedup vs baseline: 1.4081x; 1.4081x over previous
import jax
import jax.numpy as jnp
from jax import lax
from jax.experimental import pallas as pl

B, N, C, H, M = 64, 8192, 64, 64, 50
GB = 8
GRID = B // GB
NR = N // 128
NEG_INF = float("-inf")


def _split_dot(x, w):
    whi = w.astype(jnp.bfloat16).astype(jnp.float32)
    wlo = (w - whi).astype(jnp.bfloat16).astype(jnp.float32)
    x2 = jnp.concatenate([x, x], axis=1)
    w2 = jnp.concatenate([whi, wlo], axis=0)
    return lax.dot_general(x2, w2, (((1,), (0,)), ((), ())),
                           preferred_element_type=jnp.float32)


def _tree_reduce_H(h):
    chunks = [h[:, 8 * i:8 * i + 8] for i in range(8)]
    acc = chunks[0]
    for i in range(1, 8):
        acc = acc + chunks[i]
    b4 = acc[:, 0:4] + acc[:, 4:8]
    b2 = b4[:, 0:2] + b4[:, 2:4]
    return b2[:, 0:1] + b2[:, 1:2]


def _body(keys_ref, tx_ref, ty_ref, feat_ref,
          w1m_ref, b1m_ref, gm_ref, bem_ref, w2m_ref, b2m_ref,
          oc_ref, om_ref):
    iota_flat = (lax.broadcasted_iota(jnp.int32, (NR, 128), 0) * 128
                 + lax.broadcasted_iota(jnp.int32, (NR, 128), 1))
    row_iota = lax.broadcasted_iota(jnp.int32, (64, 1), 0)
    fb = feat_ref[:, 0, :].astype(jnp.bfloat16).astype(jnp.float32)

    keys = keys_ref[...]
    tx2d = tx_ref[...]
    ty2d = ty_ref[...]
    iota3 = jnp.broadcast_to(iota_flat, (GB, NR, 128))
    row3 = jnp.broadcast_to(row_iota, (GB, 64, 1))

    def step(j, carry):
        work, selx, sely = carry
        mx = jnp.max(work, axis=(1, 2), keepdims=True)
        cand = jnp.where(work == mx, iota3, jnp.int32(2 ** 31 - 1))
        ij = jnp.min(cand, axis=(1, 2), keepdims=True)
        pick = iota3 == ij
        sx = jnp.sum(jnp.where(pick, tx2d, 0.0), axis=(1, 2), keepdims=True)
        sy = jnp.sum(jnp.where(pick, ty2d, 0.0), axis=(1, 2), keepdims=True)
        work = jnp.where(pick, NEG_INF, work)
        at_j = row3 == j
        selx = jnp.where(at_j, sx, selx)
        sely = jnp.where(at_j, sy, sely)
        return work, selx, sely

    selx0 = jnp.zeros((GB, 64, 1), jnp.float32)
    _, selx, sely = lax.fori_loop(
        0, M, step, (keys, selx0, selx0), unroll=2)
    selc_all = jnp.concatenate([selx, sely], axis=2)
    sel_rows = [selc_all[bb] for bb in range(GB)]
    for bb in range(GB):
        oc_ref[bb] = sel_rows[bb][:M]

    xs = []
    for bb in range(GB):
        sb = sel_rows[bb].astype(jnp.bfloat16).astype(jnp.float32)
        xs.append(jnp.concatenate(
            [jnp.broadcast_to(fb[bb:bb + 1, :], (64, C)), sb], axis=1))
    xsel = jnp.concatenate(xs, axis=0)
    conv = _split_dot(xsel, w1m_ref[...])
    h = conv + b1m_ref[...]
    mu = _tree_reduce_H(h) * (1.0 / 64.0)
    dv = h - mu
    var = _tree_reduce_H(dv * dv) * (1.0 / 64.0)
    s = jnp.sqrt(var + 1e-5)
    hn = dv / s * gm_ref[...] + bem_ref[...]
    hr = jnp.maximum(hn, 0.0)
    lm = lax.dot_general(hr, w2m_ref[...], (((1,), (0,)), ((), ())),
                         preferred_element_type=jnp.float32) + b2m_ref[...]
    for bb in range(GB):
        om_ref[bb] = lm[bb * 64:bb * 64 + M]


def kernel(feat_in, tar_candidate, W1p, b1p, gp, bep, W2p, b2p,
           W1m, b1m, gm, bem, W2m, b2m):
    xb = jnp.concatenate(
        [jnp.broadcast_to(feat_in[:, None, :], (B, N, C)), tar_candidate],
        axis=2).astype(jnp.bfloat16)
    hp = jnp.dot(xb, W1p, preferred_element_type=jnp.float32) + b1p
    mu = jnp.mean(hp, -1, keepdims=True)
    var = jnp.mean((hp - mu) ** 2, -1, keepdims=True)
    hp = (hp - mu) / jnp.sqrt(var + 1e-5) * gp + bep
    hp = jax.nn.relu(hp)
    lp = jnp.dot(hp, W2p, preferred_element_type=jnp.float32) + b2p
    p = jax.nn.softmax(lp, axis=-1)
    keys = p[:, :, 1].reshape(B, NR, 128)
    tx3 = tar_candidate[:, :, 0].reshape(B, NR, 128)
    ty3 = tar_candidate[:, :, 1].reshape(B, NR, 128)

    feat3 = feat_in.reshape(B, 1, C)
    mspec = [
        pl.BlockSpec((66, H), lambda i: (0, 0)),
        pl.BlockSpec((H,), lambda i: (0,)),
        pl.BlockSpec((H,), lambda i: (0,)),
        pl.BlockSpec((H,), lambda i: (0,)),
        pl.BlockSpec((H, 2), lambda i: (0, 0)),
        pl.BlockSpec((2,), lambda i: (0,)),
    ]
    out = pl.pallas_call(
        _body,
        grid=(GRID,),
        in_specs=[pl.BlockSpec((GB, NR, 128), lambda i: (i, 0, 0)),
                  pl.BlockSpec((GB, NR, 128), lambda i: (i, 0, 0)),
                  pl.BlockSpec((GB, NR, 128), lambda i: (i, 0, 0)),
                  pl.BlockSpec((GB, 1, C), lambda i: (i, 0, 0))] + mspec,
        out_specs=[pl.BlockSpec((GB, M, 2), lambda i: (i, 0, 0)),
                   pl.BlockSpec((GB, M, 2), lambda i: (i, 0, 0))],
        out_shape=[jax.ShapeDtypeStruct((B, M, 2), jnp.float32),
                   jax.ShapeDtypeStruct((B, M, 2), jnp.float32)],
    )(keys, tx3, ty3, feat3, W1m, b1m, gm, bem, W2m, b2m)
    return out[0], out[1]
